# baseline (speedup 1.0000x reference)
import functools

import jax
import jax.numpy as jnp
from jax import lax
from jax.experimental import pallas as pl
from jax.experimental.pallas import tpu as pltpu
from jax.experimental.pallas import tpu_sc as plsc

FEATURE_DIM = 128
MEMORY_SIZE = 100000
N_CLASSES = 1000
BATCH = 16384

BLK = 1024
C_PAD = 1024
TC_FEAT_BLOCKS = BATCH // BLK
TC_MEM_BLOCKS = 9
TC_MEM_ROWS = TC_MEM_BLOCKS * BLK

NC = 2
NS = 16
NW = NC * NS
CHUNK = 128
SUPER = 384
KPS = SUPER // CHUNK

TAIL_START = BATCH + TC_MEM_ROWS
TAIL_ROWS = MEMORY_SIZE - TAIL_START
N_MEM_SUPERS = TAIL_ROWS // SUPER
MEM_EXTRA = N_MEM_SUPERS % NW
MAX_SUPERS_PER_W = N_MEM_SUPERS // NW + (1 if MEM_EXTRA else 0)
REM_BASE = TAIL_START + N_MEM_SUPERS * SUPER
REM = TAIL_ROWS - N_MEM_SUPERS * SUPER
assert REM in (32, 160, 288) and REM_BASE % 8 == 0


def _sc_segsum(mem_features, mem_labels):
    mesh = plsc.VectorSubcoreMesh(core_axis_name="c", subcore_axis_name="s")

    @functools.partial(
        pl.kernel,
        mesh=mesh,
        out_type=jax.ShapeDtypeStruct((NC, N_CLASSES, FEATURE_DIM), jnp.float32),
        scratch_types=[
            pltpu.VMEM((2, SUPER, FEATURE_DIM), jnp.float32),
            pltpu.VMEM((MAX_SUPERS_PER_W * KPS, CHUNK), jnp.int32),
            pltpu.VMEM((2, CHUNK), jnp.int32),
            pltpu.VMEM((32,), jnp.int32),
            pltpu.VMEM((64, FEATURE_DIM), jnp.float32),
            pltpu.VMEM_SHARED((N_CLASSES, FEATURE_DIM), jnp.float32),
            pltpu.SemaphoreType.DMA,
            pltpu.SemaphoreType.DMA,
            pltpu.SemaphoreType.DMA,
            pltpu.SemaphoreType.DMA,
            pltpu.SemaphoreType.DMA,
        ],
    )
    def k(mem_hbm, mlab_hbm, out_hbm,
          rows_v, lbuf, idx_t, idx_t32, zbuf, shared,
          semf0, semf1, sems0, sems1, seml):
        c = lax.axis_index("c")
        s = lax.axis_index("s")
        wid = s * NC + c
        sems = (semf0, semf1)
        sems_s = (sems0, sems1)

        n_t = N_MEM_SUPERS // NW + jnp.where(wid < MEM_EXTRA, 1, 0)

        def sbase(t):
            return TAIL_START + (wid + t * NW) * SUPER

        def fetch(t, b):
            pltpu.async_copy(mem_hbm.at[pl.ds(sbase(t), SUPER)],
                             rows_v.at[b], sems[b])

        def wait_fetch(b):
            pltpu.make_async_copy(mem_hbm.at[pl.ds(0, SUPER)],
                                  rows_v.at[b], sems[b]).wait()

        def scatter(t, b):
            for kk in range(KPS):
                pltpu.async_copy(rows_v.at[b, pl.ds(kk * CHUNK, CHUNK)],
                                 shared.at[lbuf.at[t * KPS + kk]], sems_s[b],
                                 add=True)

        def wait_scatter(b):
            for kk in range(KPS):
                pltpu.make_async_copy(rows_v.at[b, pl.ds(kk * CHUNK, CHUNK)],
                                      shared.at[lbuf.at[0]],
                                      sems_s[b]).wait()

        fetch(jnp.int32(0), 0)

        def lfetch(t, carry):
            for kk in range(KPS):
                pltpu.async_copy(
                    mlab_hbm.at[pl.ds(sbase(t) + kk * CHUNK, CHUNK)],
                    lbuf.at[t * KPS + kk], seml)
            return carry

        lax.fori_loop(0, n_t, lfetch, 0)

        zero16 = jnp.zeros((16,), jnp.float32)

        def zrow(r, carry):
            for cc in range(FEATURE_DIM // 16):
                zbuf[r, pl.ds(cc * 16, 16)] = zero16
            return carry

        lax.fori_loop(0, 64, zrow, 0)

        @pl.when(s < NS - 1)
        def _():
            pltpu.sync_copy(zbuf, shared.at[pl.ds(s * 64, 64)])

        @pl.when(s == NS - 1)
        def _():
            pltpu.sync_copy(zbuf.at[pl.ds(0, 40)], shared.at[pl.ds(960, 40)])

        def lwait(t, carry):
            for kk in range(KPS):
                pltpu.make_async_copy(mlab_hbm.at[pl.ds(0, CHUNK)],
                                      lbuf.at[0], seml).wait()
            return carry

        lax.fori_loop(0, n_t, lwait, 0)

        plsc.subcore_barrier()

        def step(t, carry):
            def half(b):
                wait_fetch(b)

                @pl.when(t + 1 < n_t)
                def _():
                    @pl.when(t >= 1)
                    def _():
                        wait_scatter(1 - b)

                    fetch(t + 1, 1 - b)

                scatter(t, b)

            @pl.when(t % 2 == 0)
            def _():
                half(0)

            @pl.when(t % 2 == 1)
            def _():
                half(1)

            return carry

        lax.fori_loop(0, n_t, step, 0)
        wait_scatter(0)
        wait_scatter(1)

        _off = 0
        _ci = 0
        while _off < REM:
            _sz = min(CHUNK, REM - _off)

            @pl.when(wid == 6 + _ci)
            def _(off=_off, sz=_sz, ci=_ci):
                pltpu.sync_copy(mem_hbm.at[pl.ds(REM_BASE + off, sz)],
                                rows_v.at[0, pl.ds(0, sz)])
                if sz == CHUNK:
                    pltpu.sync_copy(mlab_hbm.at[pl.ds(REM_BASE + off, sz)],
                                    idx_t.at[ci])
                    pltpu.sync_copy(rows_v.at[0, pl.ds(0, sz)],
                                    shared.at[idx_t.at[ci]], add=True)
                else:
                    pltpu.sync_copy(mlab_hbm.at[pl.ds(REM_BASE + off, sz)],
                                    idx_t32)
                    pltpu.sync_copy(rows_v.at[0, pl.ds(0, sz)],
                                    shared.at[idx_t32], add=True)

            _off += _sz
            _ci += 1

        plsc.subcore_barrier()

        @pl.when(s == 0)
        def _():
            pltpu.sync_copy(shared, out_hbm.at[c])

    return k(mem_features, mem_labels)


def _tc_body(f_ref, l_ref, m_ref, ml_ref, out_ref):
    i = pl.program_id(0)

    def accum(rows_ref, lab_ref):
        oh = (lab_ref[...].reshape(BLK, 1)
              == lax.broadcasted_iota(jnp.int32, (BLK, C_PAD), 1)
              ).astype(jnp.bfloat16)
        part = lax.dot_general(rows_ref[...].astype(jnp.bfloat16), oh,
                               (((0,), (0,)), ((), ())),
                               preferred_element_type=jnp.float32)

        @pl.when(i == 0)
        def _():
            out_ref[...] = part

        @pl.when(i > 0)
        def _():
            out_ref[...] += part

    @pl.when(i < TC_FEAT_BLOCKS)
    def _():
        accum(f_ref, l_ref)

    @pl.when(i >= TC_FEAT_BLOCKS)
    def _():
        accum(m_ref, ml_ref)


def _tc_segsum(features, labels, mem_features, mem_labels):
    nf = TC_FEAT_BLOCKS
    return pl.pallas_call(
        _tc_body,
        grid=(TC_FEAT_BLOCKS + TC_MEM_BLOCKS,),
        in_specs=[
            pl.BlockSpec((BLK, FEATURE_DIM),
                         lambda i: (jnp.minimum(i, nf - 1), 0)),
            pl.BlockSpec((BLK,),
                         lambda i: (jnp.minimum(i, nf - 1),)),
            pl.BlockSpec((BLK, FEATURE_DIM),
                         lambda i: (jnp.maximum(i, nf), 0)),
            pl.BlockSpec((BLK,),
                         lambda i: (jnp.maximum(i, nf),)),
        ],
        out_specs=pl.BlockSpec((FEATURE_DIM, C_PAD), lambda i: (0, 0)),
        out_shape=jax.ShapeDtypeStruct((FEATURE_DIM, C_PAD), jnp.float32),
    )(features, labels, mem_features, mem_labels)


def _norm_body(p_ref, t_ref, o_ref):
    tc = t_ref[...].T
    w = p_ref[0, :, :] + p_ref[1, :, :] + tc
    nrm = jnp.sqrt(jnp.sum(w * w, axis=1, keepdims=True))
    o_ref[...] = w / jnp.maximum(nrm, 1e-12)


def _combine(sc_partials, tc_partial):
    return pl.pallas_call(
        _norm_body,
        grid=(8,),
        in_specs=[
            pl.BlockSpec((2, 128, FEATURE_DIM),
                         lambda j: (0, j, 0)),
            pl.BlockSpec((FEATURE_DIM, 128), lambda j: (0, j)),
        ],
        out_specs=pl.BlockSpec((128, FEATURE_DIM),
                               lambda j: (j, 0)),
        out_shape=jax.ShapeDtypeStruct((N_CLASSES, FEATURE_DIM), jnp.float32),
    )(sc_partials, tc_partial)


def kernel(features, labels, mem_features, mem_labels):
    sc_partials = _sc_segsum(mem_features, mem_labels)
    tc_partial = _tc_segsum(features, labels, mem_features, mem_labels)
    return _combine(sc_partials, tc_partial)


# speedup vs baseline: 1.1130x; 1.1130x over previous
import functools

import jax
import jax.numpy as jnp
from jax import lax
from jax.experimental import pallas as pl
from jax.experimental.pallas import tpu as pltpu
from jax.experimental.pallas import tpu_sc as plsc

FEATURE_DIM = 128
MEMORY_SIZE = 100000
N_CLASSES = 1000
BATCH = 16384

BLK = 1024
C_PAD = 1024
TC_FEAT_BLOCKS = BATCH // BLK
TC_MEM_BLOCKS = 10
TC_MEM_ROWS = TC_MEM_BLOCKS * BLK

NC = 2
NS = 16
NW = NC * NS
CHUNK = 128
SUPER = 384
KPS = SUPER // CHUNK

TAIL_START = BATCH + TC_MEM_ROWS
TAIL_ROWS = MEMORY_SIZE - TAIL_START
N_MEM_SUPERS = TAIL_ROWS // SUPER
MEM_EXTRA = N_MEM_SUPERS % NW
MAX_SUPERS_PER_W = N_MEM_SUPERS // NW + (1 if MEM_EXTRA else 0)
REM_BASE = TAIL_START + N_MEM_SUPERS * SUPER
REM = TAIL_ROWS - N_MEM_SUPERS * SUPER
assert REM in (32, 160, 288) and REM_BASE % 8 == 0


def _sc_segsum(mem_features, mem_labels):
    mesh = plsc.VectorSubcoreMesh(core_axis_name="c", subcore_axis_name="s")

    @functools.partial(
        pl.kernel,
        mesh=mesh,
        out_type=jax.ShapeDtypeStruct((NC, N_CLASSES, FEATURE_DIM), jnp.float32),
        scratch_types=[
            pltpu.VMEM((2, SUPER, FEATURE_DIM), jnp.float32),
            pltpu.VMEM((MAX_SUPERS_PER_W * KPS, CHUNK), jnp.int32),
            pltpu.VMEM((2, CHUNK), jnp.int32),
            pltpu.VMEM((32,), jnp.int32),
            pltpu.VMEM((64, FEATURE_DIM), jnp.float32),
            pltpu.VMEM_SHARED((N_CLASSES, FEATURE_DIM), jnp.float32),
            pltpu.SemaphoreType.DMA,
            pltpu.SemaphoreType.DMA,
            pltpu.SemaphoreType.DMA,
            pltpu.SemaphoreType.DMA,
            pltpu.SemaphoreType.DMA,
        ],
    )
    def k(mem_hbm, mlab_hbm, out_hbm,
          rows_v, lbuf, idx_t, idx_t32, zbuf, shared,
          semf0, semf1, sems0, sems1, seml):
        c = lax.axis_index("c")
        s = lax.axis_index("s")
        wid = s * NC + c
        sems = (semf0, semf1)
        sems_s = (sems0, sems1)

        n_t = N_MEM_SUPERS // NW + jnp.where(wid < MEM_EXTRA, 1, 0)

        def sbase(t):
            return TAIL_START + (wid + t * NW) * SUPER

        def fetch(t, b):
            pltpu.async_copy(mem_hbm.at[pl.ds(sbase(t), SUPER)],
                             rows_v.at[b], sems[b])

        def wait_fetch(b):
            pltpu.make_async_copy(mem_hbm.at[pl.ds(0, SUPER)],
                                  rows_v.at[b], sems[b]).wait()

        def scatter(t, b):
            for kk in range(KPS):
                pltpu.async_copy(rows_v.at[b, pl.ds(kk * CHUNK, CHUNK)],
                                 shared.at[lbuf.at[t * KPS + kk]], sems_s[b],
                                 add=True)

        def wait_scatter(b):
            for kk in range(KPS):
                pltpu.make_async_copy(rows_v.at[b, pl.ds(kk * CHUNK, CHUNK)],
                                      shared.at[lbuf.at[0]],
                                      sems_s[b]).wait()

        fetch(jnp.int32(0), 0)

        def lfetch(t, carry):
            for kk in range(KPS):
                pltpu.async_copy(
                    mlab_hbm.at[pl.ds(sbase(t) + kk * CHUNK, CHUNK)],
                    lbuf.at[t * KPS + kk], seml)
            return carry

        lax.fori_loop(0, n_t, lfetch, 0)

        zero16 = jnp.zeros((16,), jnp.float32)

        def zrow(r, carry):
            for cc in range(FEATURE_DIM // 16):
                zbuf[r, pl.ds(cc * 16, 16)] = zero16
            return carry

        lax.fori_loop(0, 64, zrow, 0)

        @pl.when(s < NS - 1)
        def _():
            pltpu.sync_copy(zbuf, shared.at[pl.ds(s * 64, 64)])

        @pl.when(s == NS - 1)
        def _():
            pltpu.sync_copy(zbuf.at[pl.ds(0, 40)], shared.at[pl.ds(960, 40)])

        def lwait(t, carry):
            for kk in range(KPS):
                pltpu.make_async_copy(mlab_hbm.at[pl.ds(0, CHUNK)],
                                      lbuf.at[0], seml).wait()
            return carry

        lax.fori_loop(0, n_t, lwait, 0)

        plsc.subcore_barrier()

        def step(t, carry):
            def half(b):
                wait_fetch(b)

                @pl.when(t + 1 < n_t)
                def _():
                    @pl.when(t >= 1)
                    def _():
                        wait_scatter(1 - b)

                    fetch(t + 1, 1 - b)

                scatter(t, b)

            @pl.when(t % 2 == 0)
            def _():
                half(0)

            @pl.when(t % 2 == 1)
            def _():
                half(1)

            return carry

        lax.fori_loop(0, n_t, step, 0)
        wait_scatter(0)
        wait_scatter(1)

        _off = 0
        _ci = 0
        while _off < REM:
            _sz = min(CHUNK, REM - _off)

            @pl.when(wid == 6 + _ci)
            def _(off=_off, sz=_sz, ci=_ci):
                pltpu.sync_copy(mem_hbm.at[pl.ds(REM_BASE + off, sz)],
                                rows_v.at[0, pl.ds(0, sz)])
                if sz == CHUNK:
                    pltpu.sync_copy(mlab_hbm.at[pl.ds(REM_BASE + off, sz)],
                                    idx_t.at[ci])
                    pltpu.sync_copy(rows_v.at[0, pl.ds(0, sz)],
                                    shared.at[idx_t.at[ci]], add=True)
                else:
                    pltpu.sync_copy(mlab_hbm.at[pl.ds(REM_BASE + off, sz)],
                                    idx_t32)
                    pltpu.sync_copy(rows_v.at[0, pl.ds(0, sz)],
                                    shared.at[idx_t32], add=True)

            _off += _sz
            _ci += 1

        plsc.subcore_barrier()

        @pl.when(s == 0)
        def _():
            pltpu.sync_copy(shared, out_hbm.at[c])

    return k(mem_features, mem_labels)


def _tc_body(f_ref, l_ref, m_ref, ml_ref, out_ref):
    i = pl.program_id(0)

    def accum(rows_ref, lab_ref):
        oh = (lab_ref[...].reshape(BLK, 1)
              == lax.broadcasted_iota(jnp.int32, (BLK, C_PAD), 1)
              ).astype(jnp.bfloat16)
        part = lax.dot_general(rows_ref[...].astype(jnp.bfloat16), oh,
                               (((0,), (0,)), ((), ())),
                               preferred_element_type=jnp.float32)

        @pl.when(i == 0)
        def _():
            out_ref[...] = part

        @pl.when(i > 0)
        def _():
            out_ref[...] += part

    @pl.when(i < TC_FEAT_BLOCKS)
    def _():
        accum(f_ref, l_ref)

    @pl.when(i >= TC_FEAT_BLOCKS)
    def _():
        accum(m_ref, ml_ref)


def _tc_segsum(features, labels, mem_features, mem_labels):
    nf = TC_FEAT_BLOCKS
    return pl.pallas_call(
        _tc_body,
        grid=(TC_FEAT_BLOCKS + TC_MEM_BLOCKS,),
        in_specs=[
            pl.BlockSpec((BLK, FEATURE_DIM),
                         lambda i: (jnp.minimum(i, nf - 1), 0)),
            pl.BlockSpec((BLK,),
                         lambda i: (jnp.minimum(i, nf - 1),)),
            pl.BlockSpec((BLK, FEATURE_DIM),
                         lambda i: (jnp.maximum(i, nf), 0)),
            pl.BlockSpec((BLK,),
                         lambda i: (jnp.maximum(i, nf),)),
        ],
        out_specs=pl.BlockSpec((FEATURE_DIM, C_PAD), lambda i: (0, 0)),
        out_shape=jax.ShapeDtypeStruct((FEATURE_DIM, C_PAD), jnp.float32),
    )(features, labels, mem_features, mem_labels)


def _norm_body(p_ref, t_ref, o_ref):
    tc = t_ref[...].T[:N_CLASSES, :]
    w = p_ref[0] + p_ref[1] + tc
    nrm = jnp.sqrt(jnp.sum(w * w, axis=1, keepdims=True))
    o_ref[...] = w / jnp.maximum(nrm, 1e-12)


def _combine(sc_partials, tc_partial):
    return pl.pallas_call(
        _norm_body,
        out_shape=jax.ShapeDtypeStruct((N_CLASSES, FEATURE_DIM), jnp.float32),
    )(sc_partials, tc_partial)


def kernel(features, labels, mem_features, mem_labels):
    sc_partials = _sc_segsum(mem_features, mem_labels)
    tc_partial = _tc_segsum(features, labels, mem_features, mem_labels)
    return _combine(sc_partials, tc_partial)
